# baseline (speedup 1.0000x reference)
import functools

import jax
import jax.numpy as jnp
import numpy as np
from jax import lax
from jax.experimental import pallas as pl
from jax.experimental.pallas import tpu as pltpu
from jax.experimental.pallas import tpu_sc as plsc

VOCAB = 100000
D_MODEL = 768
B = 4
S = 2048

_NC = 2
_NS = 16
_NW = _NC * _NS

_SPW = S // _NW
_LANES = 16
_VPR = D_MODEL // _LANES


def _sinusoidal_pe(max_len, d_model):
    pos = np.arange(max_len, dtype=np.float64)[:, None]
    div = np.exp(
        np.arange(0, d_model, 2, dtype=np.float64) * -(np.log(10000.0) / d_model)
    )
    pe = np.zeros((max_len, d_model), dtype=np.float64)
    pe[:, 0::2] = np.sin(pos * div)
    pe[:, 1::2] = np.cos(pos * div)
    return pe.astype(np.float32)


_PE = _sinusoidal_pe(S, D_MODEL)


_HALF = _SPW // 2
_NCHUNK = 2 * B
_CBYTES = _HALF * D_MODEL * 4


def _sc_body(table_hbm, idx_hbm, pe_hbm, out_hbm,
             idx_v, pe_v, rows2, gsem, osem):
    wid = lax.axis_index("s") * _NC + lax.axis_index("c")
    s0 = wid * _SPW

    for b in range(B):
        pltpu.sync_copy(idx_hbm.at[b, pl.ds(s0, _SPW)],
                        idx_v.at[pl.ds(b * _SPW, _SPW)])

    def issue_gather(k):
        p = jnp.bitwise_and(k, 1)
        pltpu.async_copy(
            table_hbm.at[idx_v.at[pl.ds(k * _HALF, _HALF)]],
            rows2.at[p], gsem)

    issue_gather(0)
    pltpu.sync_copy(pe_hbm.at[pl.ds(s0 * D_MODEL, _SPW * D_MODEL)], pe_v)

    def chunk(k, _):
        p = jnp.bitwise_and(k, 1)

        @pl.when(k >= 1)
        def _():
            pltpu.make_async_copy(
                rows2.at[1 - p], out_hbm.at[0, pl.ds(s0, _HALF)], osem
            ).wait()

        @pl.when(k < _NCHUNK - 1)
        def _():
            issue_gather(k + 1)

        pltpu.make_async_copy(
            table_hbm.at[idx_v.at[pl.ds(k * _HALF, _HALF)]],
            rows2.at[p], gsem).wait()

        def row_add(r, _):
            pbase = (p * _HALF + r) * D_MODEL
            for g in range(0, _VPR, 8):
                vals = [
                    pe_v[pl.ds(pbase + (g + j) * _LANES, _LANES)]
                    for j in range(8)
                ]
                for j in range(8):
                    plsc.addupdate(
                        rows2.at[p, r, pl.ds((g + j) * _LANES, _LANES)],
                        vals[j],
                    )
            return ()

        lax.fori_loop(0, _HALF, row_add, (), unroll=False)

        b = lax.shift_right_logical(k, 1)
        pltpu.async_copy(
            rows2.at[p], out_hbm.at[b, pl.ds(s0 + p * _HALF, _HALF)], osem)
        return ()

    lax.fori_loop(0, _NCHUNK, chunk, (), unroll=False)
    pltpu.make_async_copy(
        rows2.at[1], out_hbm.at[0, pl.ds(s0, _HALF)], osem).wait()


@jax.jit
def _embed(x, table, pe):
    idx = x.astype(jnp.int32)
    mesh = plsc.VectorSubcoreMesh(core_axis_name="c", subcore_axis_name="s")
    out = pl.kernel(
        _sc_body,
        out_type=jax.ShapeDtypeStruct((B, S, D_MODEL), jnp.float32),
        mesh=mesh,
        scratch_types=[
            pltpu.VMEM((B * _SPW,), jnp.int32),
            pltpu.VMEM((_SPW * D_MODEL,), jnp.float32),
            pltpu.VMEM((2, _HALF, D_MODEL), jnp.float32),
            pltpu.SemaphoreType.DMA,
            pltpu.SemaphoreType.DMA,
        ],
    )(table, idx, pe)
    return out


_PE_DEV = None


def kernel(x, table):
    global _PE_DEV
    if _PE_DEV is None:
        _PE_DEV = jnp.asarray(_PE.reshape(-1))
    return _embed(x, table, _PE_DEV)


# speedup vs baseline: 1.1245x; 1.1245x over previous
import functools

import jax
import jax.numpy as jnp
import numpy as np
from jax import lax
from jax.experimental import pallas as pl
from jax.experimental.pallas import tpu as pltpu
from jax.experimental.pallas import tpu_sc as plsc

VOCAB = 100000
D_MODEL = 768
B = 4
S = 2048

_NC = 2
_NS = 16
_NW = _NC * _NS

_SPW = S // _NW
_LANES = 16
_VPR = D_MODEL // _LANES
_WPR = D_MODEL // 2


def _sinusoidal_pe(max_len, d_model):
    pos = np.arange(max_len, dtype=np.float64)[:, None]
    div = np.exp(
        np.arange(0, d_model, 2, dtype=np.float64) * -(np.log(10000.0) / d_model)
    )
    pe = np.zeros((max_len, d_model), dtype=np.float64)
    pe[:, 0::2] = np.sin(pos * div)
    pe[:, 1::2] = np.cos(pos * div)
    return pe.astype(np.float32)


def _pe_bf16_shuffled():
    import ml_dtypes

    pe = _sinusoidal_pe(S, D_MODEL)
    pe3 = pe.reshape(S, D_MODEL // 32, 2, 16)
    ilv = np.stack([pe3[:, :, 0, :], pe3[:, :, 1, :]], axis=-1)
    flat = np.ascontiguousarray(
        ilv.reshape(S * D_MODEL)).astype(ml_dtypes.bfloat16)
    return flat.view(np.int32)


_PE = _pe_bf16_shuffled()


_HALF = _SPW // 2
_NCHUNK = 2 * B
_CBYTES = _HALF * D_MODEL * 4


def _sc_body(table_hbm, idx_hbm, pe_hbm, out_hbm,
             idx_v, pe_v, rows2, gsem, osem):
    wid = lax.axis_index("s") * _NC + lax.axis_index("c")
    s0 = wid * _SPW

    for b in range(B):
        pltpu.sync_copy(idx_hbm.at[b, pl.ds(s0, _SPW)],
                        idx_v.at[pl.ds(b * _SPW, _SPW)])

    def issue_gather(k):
        p = jnp.bitwise_and(k, 1)
        pltpu.async_copy(
            table_hbm.at[idx_v.at[pl.ds(k * _HALF, _HALF)]],
            rows2.at[p], gsem)

    issue_gather(0)
    pltpu.sync_copy(
        pe_hbm.at[pl.ds(s0 * _WPR, _SPW * _WPR)], pe_v)

    def chunk(k, _):
        p = jnp.bitwise_and(k, 1)

        @pl.when(k >= 1)
        def _():
            pltpu.make_async_copy(
                rows2.at[1 - p], out_hbm.at[0, pl.ds(s0, _HALF)], osem
            ).wait()

        @pl.when(k < _NCHUNK - 1)
        def _():
            issue_gather(k + 1)

        pltpu.make_async_copy(
            table_hbm.at[idx_v.at[pl.ds(k * _HALF, _HALF)]],
            rows2.at[p], gsem).wait()

        @plsc.parallel_loop(0, _HALF, 1)
        def row_add(r):
            pbase = (p * _HALF + r) * _WPR
            for g2 in range(0, _VPR // 2, 4):
                vals = [
                    pe_v[pl.ds(pbase + (g2 + j) * _LANES, _LANES)]
                    for j in range(4)
                ]
                for j in range(4):
                    w = vals[j]
                    a = lax.bitcast_convert_type(
                        lax.shift_left(w, jnp.int32(16)), jnp.float32)
                    b = lax.bitcast_convert_type(
                        jnp.bitwise_and(w, jnp.int32(-65536)), jnp.float32)
                    plsc.addupdate(
                        rows2.at[p, r, pl.ds((g2 + j) * 32, _LANES)], a)
                    plsc.addupdate(
                        rows2.at[p, r, pl.ds((g2 + j) * 32 + 16, _LANES)], b)

        b = lax.shift_right_logical(k, 1)
        pltpu.async_copy(
            rows2.at[p], out_hbm.at[b, pl.ds(s0 + p * _HALF, _HALF)], osem)
        return ()

    lax.fori_loop(0, _NCHUNK, chunk, (), unroll=False)
    pltpu.make_async_copy(
        rows2.at[1], out_hbm.at[0, pl.ds(s0, _HALF)], osem).wait()


@jax.jit
def _embed(x, table, pe):
    idx = x.astype(jnp.int32)
    mesh = plsc.VectorSubcoreMesh(core_axis_name="c", subcore_axis_name="s")
    out = pl.kernel(
        _sc_body,
        out_type=jax.ShapeDtypeStruct((B, S, D_MODEL), jnp.float32),
        mesh=mesh,
        scratch_types=[
            pltpu.VMEM((B * _SPW,), jnp.int32),
            pltpu.VMEM((_SPW * _WPR,), jnp.int32),
            pltpu.VMEM((2, _HALF, D_MODEL), jnp.float32),
            pltpu.SemaphoreType.DMA,
            pltpu.SemaphoreType.DMA,
        ],
    )(table, idx, pe)
    return out


_PE_DEV = None


def kernel(x, table):
    global _PE_DEV
    if _PE_DEV is None:
        _PE_DEV = jnp.asarray(_PE)
    return _embed(x, table, _PE_DEV)
